# baseline (speedup 1.0000x reference)
import functools

import jax
import jax.numpy as jnp
from jax import lax
from jax.experimental import pallas as pl
from jax.experimental.pallas import tpu as pltpu
from jax.experimental.pallas import tpu_sc as plsc

N = 4096
D = 128
ROW_BLK = 256
GRID = N // ROW_BLK

NC = 2
NS = 16
NW = NC * NS
B_PER_W = N // NW
LANES = 16


def _degree_body(adj_ref, outd_ref, ind_ref):
    i = pl.program_id(0)
    blk = adj_ref[...]
    outd_ref[...] = jnp.sum(blk, axis=1).astype(jnp.int32)
    col = jnp.sum(blk, axis=0).astype(jnp.int32)

    @pl.when(i == 0)
    def _():
        ind_ref[...] = col

    @pl.when(i > 0)
    def _():
        ind_ref[...] = ind_ref[...] + col


def _degrees(adj):
    return pl.pallas_call(
        _degree_body,
        grid=(GRID,),
        in_specs=[pl.BlockSpec((ROW_BLK, N), lambda i: (i, 0))],
        out_specs=[
            pl.BlockSpec((ROW_BLK,), lambda i: (i,)),
            pl.BlockSpec((N,), lambda i: (0,)),
        ],
        out_shape=[
            jax.ShapeDtypeStruct((N,), jnp.int32),
            jax.ShapeDtypeStruct((N,), jnp.int32),
        ],
    )(adj)


_SC_MESH = plsc.VectorSubcoreMesh(core_axis_name="c", subcore_axis_name="s")


@functools.partial(
    pl.kernel,
    mesh=_SC_MESH,
    out_type=jax.ShapeDtypeStruct((N, D), jnp.float32),
    scratch_types=[
        pltpu.VMEM((B_PER_W,), jnp.int32),
        pltpu.VMEM((B_PER_W,), jnp.int32),
        pltpu.VMEM((B_PER_W, D), jnp.float32),
        pltpu.VMEM((B_PER_W, D), jnp.float32),
        pltpu.SemaphoreType.DMA,
        pltpu.SemaphoreType.DMA,
    ],
)
def _gather_add(ind_hbm, outd_hbm, in_tab, out_tab, out_hbm,
                idx_i, idx_o, rows_i, rows_o, sem_i, sem_o):
    wid = lax.axis_index("s") * NC + lax.axis_index("c")
    base = wid * B_PER_W
    li = pltpu.async_copy(ind_hbm.at[pl.ds(base, B_PER_W)], idx_i, sem_i)
    lo = pltpu.async_copy(outd_hbm.at[pl.ds(base, B_PER_W)], idx_o, sem_o)
    li.wait()
    lo.wait()
    ci = pltpu.async_copy(in_tab.at[idx_i], rows_i, sem_i)
    co = pltpu.async_copy(out_tab.at[idx_o], rows_o, sem_o)
    ci.wait()
    co.wait()

    def body(r, carry):
        for c in range(D // LANES):
            sl = (r, pl.ds(c * LANES, LANES))
            rows_i[sl] = rows_i[sl] + rows_o[sl]
        return carry

    lax.fori_loop(0, B_PER_W, body, 0)
    pltpu.sync_copy(rows_i, out_hbm.at[pl.ds(base, B_PER_W)])


def kernel(dense_adj_mx, in_table, out_table):
    outdegree, indegree = _degrees(dense_adj_mx)
    return _gather_add(indegree, outdegree, in_table, out_table)


# speedup vs baseline: 1.0455x; 1.0455x over previous
import functools

import jax
import jax.numpy as jnp
from jax import lax
from jax.experimental import pallas as pl
from jax.experimental.pallas import tpu as pltpu
from jax.experimental.pallas import tpu_sc as plsc

N = 4096
D = 128
ROW_BLK = 1024
GRID = N // ROW_BLK

NC = 2
NS = 16
NW = NC * NS
B_PER_W = N // NW
LANES = 16


def _degree_body(adj_ref, outd_ref, ind_ref):
    i = pl.program_id(0)
    blk = adj_ref[...]
    outd_ref[...] = jnp.sum(blk, axis=1).astype(jnp.int32)
    col = jnp.sum(blk, axis=0).astype(jnp.int32)

    @pl.when(i == 0)
    def _():
        ind_ref[...] = col

    @pl.when(i > 0)
    def _():
        ind_ref[...] = ind_ref[...] + col


def _degrees(adj):
    return pl.pallas_call(
        _degree_body,
        grid=(GRID,),
        in_specs=[pl.BlockSpec((ROW_BLK, N), lambda i: (i, 0))],
        out_specs=[
            pl.BlockSpec((ROW_BLK,), lambda i: (i,)),
            pl.BlockSpec((N,), lambda i: (0,)),
        ],
        out_shape=[
            jax.ShapeDtypeStruct((N,), jnp.int32),
            jax.ShapeDtypeStruct((N,), jnp.int32),
        ],
    )(adj)


_SC_MESH = plsc.VectorSubcoreMesh(core_axis_name="c", subcore_axis_name="s")


@functools.partial(
    pl.kernel,
    mesh=_SC_MESH,
    out_type=jax.ShapeDtypeStruct((N, D), jnp.float32),
    scratch_types=[
        pltpu.VMEM((B_PER_W,), jnp.int32),
        pltpu.VMEM((B_PER_W,), jnp.int32),
        pltpu.VMEM((B_PER_W, D), jnp.float32),
        pltpu.VMEM((B_PER_W, D), jnp.float32),
        pltpu.SemaphoreType.DMA,
        pltpu.SemaphoreType.DMA,
    ],
)
def _gather_add(ind_hbm, outd_hbm, in_tab, out_tab, out_hbm,
                idx_i, idx_o, rows_i, rows_o, sem_i, sem_o):
    wid = lax.axis_index("s") * NC + lax.axis_index("c")
    base = wid * B_PER_W
    li = pltpu.async_copy(ind_hbm.at[pl.ds(base, B_PER_W)], idx_i, sem_i)
    lo = pltpu.async_copy(outd_hbm.at[pl.ds(base, B_PER_W)], idx_o, sem_o)
    li.wait()
    lo.wait()
    ci = pltpu.async_copy(in_tab.at[idx_i], rows_i, sem_i)
    co = pltpu.async_copy(out_tab.at[idx_o], rows_o, sem_o)
    ci.wait()
    co.wait()

    def body(r, carry):
        for c in range(D // LANES):
            sl = (r, pl.ds(c * LANES, LANES))
            rows_i[sl] = rows_i[sl] + rows_o[sl]
        return carry

    lax.fori_loop(0, B_PER_W, body, 0)
    pltpu.sync_copy(rows_i, out_hbm.at[pl.ds(base, B_PER_W)])


def kernel(dense_adj_mx, in_table, out_table):
    outdegree, indegree = _degrees(dense_adj_mx)
    return _gather_add(indegree, outdegree, in_table, out_table)


# speedup vs baseline: 1.0924x; 1.0448x over previous
import functools

import jax
import jax.numpy as jnp
from jax import lax
from jax.experimental import pallas as pl
from jax.experimental.pallas import tpu as pltpu
from jax.experimental.pallas import tpu_sc as plsc

N = 4096
D = 128
ROW_BLK = 512
GRID = N // ROW_BLK
HALF = N // 2

NC = 2
NS = 16
NW = NC * NS
B_PER_W = N // NW
LANES = 16


def _degree_body(adj_l_ref, adj_r_ref, outd_ref, ind_ref):
    i = pl.program_id(0)
    blk_l = adj_l_ref[...]
    blk_r = adj_r_ref[...]
    outd_ref[...] = (jnp.sum(blk_l, axis=1)
                     + jnp.sum(blk_r, axis=1)).astype(jnp.int32)
    col_l = jnp.sum(blk_l, axis=0).astype(jnp.int32)
    col_r = jnp.sum(blk_r, axis=0).astype(jnp.int32)

    @pl.when(i == 0)
    def _():
        ind_ref[pl.ds(0, HALF)] = col_l
        ind_ref[pl.ds(HALF, HALF)] = col_r

    @pl.when(i > 0)
    def _():
        ind_ref[pl.ds(0, HALF)] = ind_ref[pl.ds(0, HALF)] + col_l
        ind_ref[pl.ds(HALF, HALF)] = ind_ref[pl.ds(HALF, HALF)] + col_r


def _degrees(adj):
    return pl.pallas_call(
        _degree_body,
        grid=(GRID,),
        in_specs=[
            pl.BlockSpec((ROW_BLK, HALF), lambda i: (i, 0)),
            pl.BlockSpec((ROW_BLK, HALF), lambda i: (i, 1)),
        ],
        out_specs=[
            pl.BlockSpec((ROW_BLK,), lambda i: (i,)),
            pl.BlockSpec((N,), lambda i: (0,)),
        ],
        out_shape=[
            jax.ShapeDtypeStruct((N,), jnp.int32),
            jax.ShapeDtypeStruct((N,), jnp.int32),
        ],
    )(adj, adj)


_SC_MESH = plsc.VectorSubcoreMesh(core_axis_name="c", subcore_axis_name="s")


@functools.partial(
    pl.kernel,
    mesh=_SC_MESH,
    out_type=jax.ShapeDtypeStruct((N, D), jnp.float32),
    scratch_types=[
        pltpu.VMEM((B_PER_W,), jnp.int32),
        pltpu.VMEM((B_PER_W,), jnp.int32),
        pltpu.VMEM((B_PER_W, D), jnp.float32),
        pltpu.VMEM((B_PER_W, D), jnp.float32),
        pltpu.SemaphoreType.DMA,
        pltpu.SemaphoreType.DMA,
    ],
)
def _gather_add(ind_hbm, outd_hbm, in_tab, out_tab, out_hbm,
                idx_i, idx_o, rows_i, rows_o, sem_i, sem_o):
    wid = lax.axis_index("s") * NC + lax.axis_index("c")
    base = wid * B_PER_W
    li = pltpu.async_copy(ind_hbm.at[pl.ds(base, B_PER_W)], idx_i, sem_i)
    lo = pltpu.async_copy(outd_hbm.at[pl.ds(base, B_PER_W)], idx_o, sem_o)
    li.wait()
    lo.wait()
    ci = pltpu.async_copy(in_tab.at[idx_i], rows_i, sem_i)
    co = pltpu.async_copy(out_tab.at[idx_o], rows_o, sem_o)
    ci.wait()
    co.wait()

    def body(r, carry):
        for c in range(D // LANES):
            sl = (r, pl.ds(c * LANES, LANES))
            rows_i[sl] = rows_i[sl] + rows_o[sl]
        return carry

    lax.fori_loop(0, B_PER_W, body, 0)
    pltpu.sync_copy(rows_i, out_hbm.at[pl.ds(base, B_PER_W)])


def kernel(dense_adj_mx, in_table, out_table):
    outdegree, indegree = _degrees(dense_adj_mx)
    return _gather_add(indegree, outdegree, in_table, out_table)
